# baseline (speedup 1.0000x reference)
import functools

import jax
import jax.numpy as jnp
from jax import lax
from jax.experimental import pallas as pl
from jax.experimental.pallas import tpu as pltpu
from jax.experimental.pallas import tpu_sc as plsc

IN_F = 1000000
OUT_F = 32
BATCH = 16384

NC = 2
NS = 16
L = 16
NW = NC * NS
BPW = BATCH // NW
CHUNK = 128
NCHUNK = BPW // CHUNK
ROWS_PER_STEP = 8

_mesh = plsc.VectorSubcoreMesh(core_axis_name="c", subcore_axis_name="s")


@functools.partial(
    pl.kernel,
    mesh=_mesh,
    compiler_params=pltpu.CompilerParams(use_tc_tiling_on_sc=False),
    out_type=jax.ShapeDtypeStruct((BATCH, OUT_F), jnp.float32),
    scratch_types=[
        pltpu.VMEM((NCHUNK, CHUNK), jnp.int32),
        pltpu.VMEM((BPW, OUT_F), jnp.float32),
        pltpu.VMEM((OUT_F,), jnp.float32),
        pltpu.SemaphoreType.DMA,
    ],
)
def _gather_bias(idx_hbm, table_hbm, bias_hbm, out_hbm, idx_v, rows_v, bias_v, sem):
    wid = lax.axis_index("s") * NC + lax.axis_index("c")
    pltpu.sync_copy(idx_hbm.at[pl.ds(wid * NCHUNK, NCHUNK)], idx_v)
    pltpu.sync_copy(bias_hbm, bias_v)
    copies = [
        pltpu.async_copy(
            table_hbm.at[idx_v.at[j]],
            rows_v.at[pl.ds(j * CHUNK, CHUNK)],
            sem,
        )
        for j in range(NCHUNK)
    ]
    for c in copies:
        c.wait()
    b0 = bias_v[pl.ds(0, L)]
    b1 = bias_v[pl.ds(L, L)]

    def body(i, carry):
        r0 = i * ROWS_PER_STEP
        for k in range(ROWS_PER_STEP):
            rows_v[r0 + k, pl.ds(0, L)] = rows_v[r0 + k, pl.ds(0, L)] + b0
            rows_v[r0 + k, pl.ds(L, L)] = rows_v[r0 + k, pl.ds(L, L)] + b1
        return carry

    lax.fori_loop(0, BPW // ROWS_PER_STEP, body, 0)
    pltpu.sync_copy(rows_v, out_hbm.at[pl.ds(wid * BPW, BPW)])


def kernel(input, weight, bias):
    idx = input.astype(jnp.int32).reshape(NW * NCHUNK, CHUNK)
    return _gather_bias(idx, weight, bias)
